# baseline (speedup 1.0000x reference)
import functools

import jax
import jax.numpy as jnp
from jax import lax
from jax.experimental import pallas as pl
from jax.experimental.pallas import tpu as pltpu
from jax.experimental.pallas import tpu_sc as plsc

VOCAB = 100000
HID = 128
B = 1024
S = 200
EPS = 1e-12

NW = 32
SLABS = (128, 768, 128)
SLAB_CHUNK = {128: 80, 768: 120}
BB = 32


def _gather_body(tok_per_w, nchunk, CHUNK):
    def _gather_kernel(ids_hbm, table_hbm, out_hbm, idx_v, buf0, buf1,
                       sem0, sem1):
        wid = lax.axis_index("s") * 2 + lax.axis_index("c")
        base = wid * tok_per_w
        pltpu.sync_copy(ids_hbm.at[wid], idx_v)
        bufs = (buf0, buf1)
        sems = (sem0, sem1)

        def start(c):
            return pltpu.async_copy(
                table_hbm.at[idx_v.at[pl.ds(c * CHUNK, CHUNK)]],
                bufs[c % 2], sems[c % 2])

        handles = [None] * nchunk
        handles[0] = start(0)
        for c in range(nchunk):
            if c + 1 < nchunk:
                handles[c + 1] = start(c + 1)
            handles[c].wait()
            pltpu.sync_copy(bufs[c % 2],
                            out_hbm.at[pl.ds(base + c * CHUNK, CHUNK)])

    return _gather_kernel


def _sc_gather(ids, word_emb, sb):
    stok = sb * S
    tok_per_w = stok // NW
    CHUNK = SLAB_CHUNK[sb]
    nchunk = tok_per_w // CHUNK
    mesh = plsc.VectorSubcoreMesh(core_axis_name="c", subcore_axis_name="s")
    k = functools.partial(
        pl.kernel,
        mesh=mesh,
        out_type=jax.ShapeDtypeStruct((stok, HID), jnp.float32),
        scratch_types=[
            pltpu.VMEM((tok_per_w,), jnp.int32),
            pltpu.VMEM((CHUNK, HID), jnp.float32),
            pltpu.VMEM((CHUNK, HID), jnp.float32),
            pltpu.SemaphoreType.DMA,
            pltpu.SemaphoreType.DMA,
        ],
    )(_gather_body(tok_per_w, nchunk, CHUNK))
    return k(ids, word_emb)


def _ln_kernel(words_ref, tt_ref, pt0_ref, ptd_ref, gamma_ref, beta_ref,
               prev_ref, out_ref):
    del prev_ref
    words = words_ref[...]
    tt = tt_ref[:, 0, :].astype(jnp.float32)
    emb = (words + pt0_ref[...][None, :, :]
           + tt[:, :, None] * ptd_ref[0][None, None, :])
    mu = jnp.mean(emb, axis=-1, keepdims=True)
    xc = emb - mu
    var = jnp.mean(xc * xc, axis=-1, keepdims=True)
    y = xc * lax.rsqrt(var + EPS)
    out_ref[...] = y * gamma_ref[0][None, None, :] + beta_ref[0][None, None, :]


def _tc_add_ln(step_off, steps, words, token_type_ids, pt0, ptd, gamma, beta,
               prev):
    in_specs = [
        pl.BlockSpec((BB, S, HID), lambda i: (i, 0, 0)),
        pl.BlockSpec((BB, 1, S), lambda i: (i, 0, 0)),
        pl.BlockSpec((S, HID), lambda i: (0, 0)),
        pl.BlockSpec((1, HID), lambda i: (0, 0)),
        pl.BlockSpec((1, HID), lambda i: (0, 0)),
        pl.BlockSpec((1, HID), lambda i: (0, 0)),
    ]
    args = [words, token_type_ids, pt0, ptd, gamma, beta]
    aliases = {}
    body = _ln_kernel
    if prev is not None:
        in_specs.append(pl.BlockSpec(memory_space=pl.ANY))
        args.append(prev)
        aliases = {6: 0}
    else:
        body = functools.partial(
            lambda *refs: _ln_kernel(*refs[:6], None, refs[6]))
    return pl.pallas_call(
        body,
        grid=(steps,),
        in_specs=in_specs,
        out_specs=pl.BlockSpec(
            (BB, S, HID), lambda i, _o=step_off: (_o + i, 0, 0)),
        out_shape=jax.ShapeDtypeStruct((B, S, HID), jnp.float32),
        input_output_aliases=aliases,
    )(*args)


def kernel(input_ids, token_type_ids, word_emb, pos_emb, type_emb, gamma, beta):
    ids_flat = input_ids.astype(jnp.int32).reshape(-1)
    tt = token_type_ids.astype(jnp.int32).reshape(B, 1, S)
    pt0 = pos_emb[:S] + type_emb[0][None, :]
    ptd = (type_emb[1] - type_emb[0]).reshape(1, HID)
    g2 = gamma.reshape(1, HID)
    b2 = beta.reshape(1, HID)
    slab_words = []
    row = 0
    for sb in SLABS:
        stok = sb * S
        ids_s = lax.dynamic_slice_in_dim(ids_flat, row * S, stok).reshape(
            NW, stok // NW)
        slab_words.append(_sc_gather(ids_s, word_emb, sb).reshape(sb, S, HID))
        row += sb
    out = None
    row = 0
    for sb, words in zip(SLABS, slab_words):
        out = _tc_add_ln(row // BB, sb // BB, words,
                         tt[row:row + sb], pt0, ptd, g2, b2, out)
        row += sb
    return out.reshape(B, S, HID)


# speedup vs baseline: 1.0609x; 1.0609x over previous
import functools

import jax
import jax.numpy as jnp
from jax import lax
from jax.experimental import pallas as pl
from jax.experimental.pallas import tpu as pltpu
from jax.experimental.pallas import tpu_sc as plsc

VOCAB = 100000
HID = 128
B = 1024
S = 200
EPS = 1e-12

NW = 32
NSLAB = 2
SB = B // NSLAB
STOK = SB * S
TOK_PER_W = STOK // NW
CHUNK = 128
NCHUNK = TOK_PER_W // CHUNK
BB = 32
SLAB_STEPS = SB // BB


def _gather_kernel(ids_hbm, table_hbm, out_hbm, idx_v, buf0, buf1, sem0, sem1):
    wid = lax.axis_index("s") * 2 + lax.axis_index("c")
    base = wid * TOK_PER_W
    pltpu.sync_copy(ids_hbm.at[wid], idx_v)
    bufs = (buf0, buf1)
    sems = (sem0, sem1)

    def start(c):
        return pltpu.async_copy(
            table_hbm.at[idx_v.at[pl.ds(c * CHUNK, CHUNK)]],
            bufs[c % 2], sems[c % 2])

    handles = [None] * NCHUNK
    handles[0] = start(0)
    for c in range(NCHUNK):
        if c + 1 < NCHUNK:
            handles[c + 1] = start(c + 1)
        handles[c].wait()
        pltpu.sync_copy(bufs[c % 2], out_hbm.at[pl.ds(base + c * CHUNK, CHUNK)])


def _sc_gather(ids, word_emb):
    mesh = plsc.VectorSubcoreMesh(core_axis_name="c", subcore_axis_name="s")
    k = functools.partial(
        pl.kernel,
        mesh=mesh,
        out_type=jax.ShapeDtypeStruct((STOK, HID), jnp.float32),
        scratch_types=[
            pltpu.VMEM((TOK_PER_W,), jnp.int32),
            pltpu.VMEM((CHUNK, HID), jnp.float32),
            pltpu.VMEM((CHUNK, HID), jnp.float32),
            pltpu.SemaphoreType.DMA,
            pltpu.SemaphoreType.DMA,
        ],
    )(_gather_kernel)
    return k(ids, word_emb)


def _ln_kernel(words_ref, tt_ref, pt0_ref, ptd_ref, gamma_ref, beta_ref,
               prev_ref, out_ref):
    del prev_ref
    words = words_ref[...]
    tt = tt_ref[:, 0, :].astype(jnp.float32)
    emb = (words + pt0_ref[...][None, :, :]
           + tt[:, :, None] * ptd_ref[0][None, None, :])
    mu = jnp.mean(emb, axis=-1, keepdims=True)
    xc = emb - mu
    var = jnp.mean(xc * xc, axis=-1, keepdims=True)
    y = xc * lax.rsqrt(var + EPS)
    out_ref[...] = y * gamma_ref[0][None, None, :] + beta_ref[0][None, None, :]


def _tc_add_ln(slab, words, token_type_ids, pt0, ptd, gamma, beta, prev):
    in_specs = [
        pl.BlockSpec((BB, S, HID), lambda i: (i, 0, 0)),
        pl.BlockSpec((BB, 1, S), lambda i: (i, 0, 0)),
        pl.BlockSpec((S, HID), lambda i: (0, 0)),
        pl.BlockSpec((1, HID), lambda i: (0, 0)),
        pl.BlockSpec((1, HID), lambda i: (0, 0)),
        pl.BlockSpec((1, HID), lambda i: (0, 0)),
    ]
    args = [words, token_type_ids, pt0, ptd, gamma, beta]
    aliases = {}
    body = _ln_kernel
    if prev is not None:
        in_specs.append(pl.BlockSpec(memory_space=pl.ANY))
        args.append(prev)
        aliases = {6: 0}
    else:
        body = functools.partial(
            lambda *refs: _ln_kernel(*refs[:6], None, refs[6]))
    return pl.pallas_call(
        body,
        grid=(SLAB_STEPS,),
        in_specs=in_specs,
        out_specs=pl.BlockSpec(
            (BB, S, HID), lambda i, _s=slab: (_s * SLAB_STEPS + i, 0, 0)),
        out_shape=jax.ShapeDtypeStruct((B, S, HID), jnp.float32),
        input_output_aliases=aliases,
    )(*args)


def kernel(input_ids, token_type_ids, word_emb, pos_emb, type_emb, gamma, beta):
    ids = input_ids.astype(jnp.int32).reshape(NSLAB, NW, TOK_PER_W)
    tt = token_type_ids.astype(jnp.int32).reshape(NSLAB, SB, 1, S)
    pt0 = pos_emb[:S] + type_emb[0][None, :]
    ptd = (type_emb[1] - type_emb[0]).reshape(1, HID)
    g2 = gamma.reshape(1, HID)
    b2 = beta.reshape(1, HID)
    slab_words = [
        _sc_gather(ids[s], word_emb).reshape(SB, S, HID) for s in range(NSLAB)
    ]
    out = None
    for s in range(NSLAB):
        out = _tc_add_ln(s, slab_words[s], tt[s], pt0, ptd, g2, b2, out)
    return out.reshape(B, S, HID)
